# baseline (speedup 1.0000x reference)
import functools

import jax
import jax.numpy as jnp
from jax.experimental import pallas as pl
from jax.experimental.pallas import tpu as pltpu

E = 8
TOP_K = 2
D = 2048
F = 1408
N_SHARED = 2
NE = E + N_SHARED


def _gate_kernel(x_ref, gw_ref, comb_ref):
    x = x_ref[...]
    logits = jax.lax.dot_general(
        x, gw_ref[...], (((1,), (1,)), ((), ())),
        preferred_element_type=jnp.float32)
    tb = logits.shape[0]
    lane = jax.lax.broadcasted_iota(jnp.int32, (tb, E), 1)
    neg = jnp.float32(-jnp.inf)
    m1 = jnp.max(logits, axis=1, keepdims=True)
    e1 = jnp.min(jnp.where(logits == m1, lane, E), axis=1, keepdims=True)
    masked = jnp.where(lane == e1, neg, logits)
    m2 = jnp.max(masked, axis=1, keepdims=True)
    e2 = jnp.min(jnp.where(masked == m2, lane, E), axis=1, keepdims=True)
    w1 = 1.0 / (1.0 + jnp.exp(m2 - m1))
    w2 = 1.0 - w1
    lane16 = jax.lax.broadcasted_iota(jnp.int32, (tb, 2 * E), 1)
    comb = jnp.where(lane16 == e1, w1, 0.0) + jnp.where(lane16 == e2, w2, 0.0)
    comb = comb + jnp.where((lane16 >= E) & (lane16 < NE), 1.0, 0.0)
    comb_ref[...] = comb


def _moe_kernel(comb_ref, x_ref, gw_ref, uw_ref, dw_ref, o_ref):
    e = pl.program_id(1)
    x = x_ref[...]
    g = jnp.dot(x, gw_ref[0], preferred_element_type=jnp.float32)
    u = jnp.dot(x, uw_ref[0], preferred_element_type=jnp.float32)
    h = (g * jax.lax.logistic(g) * u).astype(jnp.bfloat16)
    z = jnp.dot(h, dw_ref[0], preferred_element_type=jnp.float32)
    tb = z.shape[0]
    lane16 = jax.lax.broadcasted_iota(jnp.int32, (tb, 2 * E), 1)
    w = jnp.sum(jnp.where(lane16 == e, comb_ref[...], 0.0), axis=1,
                keepdims=True)
    contrib = w * z

    @pl.when(e == 0)
    def _():
        o_ref[...] = contrib

    @pl.when(e > 0)
    def _():
        o_ref[...] += contrib


@jax.jit
def kernel(hidden_states, gate_weight, expert_gate_w, expert_up_w,
           expert_down_w, shared_gate_w, shared_up_w, shared_down_w):
    orig_shape = hidden_states.shape
    x = hidden_states.reshape(-1, D)
    t = x.shape[0]
    tb = min(t, 512)

    comb = pl.pallas_call(
        _gate_kernel,
        grid=(t // tb,),
        in_specs=[
            pl.BlockSpec((tb, D), lambda i: (i, 0)),
            pl.BlockSpec((E, D), lambda i: (0, 0)),
        ],
        out_specs=pl.BlockSpec((tb, 2 * E), lambda i: (i, 0)),
        out_shape=jax.ShapeDtypeStruct((t, 2 * E), jnp.float32),
    )(x, gate_weight)

    bf = jnp.bfloat16
    gws = jnp.concatenate(
        [expert_gate_w,
         shared_gate_w.reshape(D, N_SHARED, F).transpose(1, 0, 2)],
        axis=0).astype(bf)
    uws = jnp.concatenate(
        [expert_up_w,
         shared_up_w.reshape(D, N_SHARED, F).transpose(1, 0, 2)],
        axis=0).astype(bf)
    dws = jnp.concatenate(
        [expert_down_w, shared_down_w.reshape(N_SHARED, F, D)],
        axis=0).astype(bf)
    xb = x.astype(bf)

    y = pl.pallas_call(
        _moe_kernel,
        grid=(t // tb, NE),
        in_specs=[
            pl.BlockSpec((tb, 2 * E), lambda i, e: (i, 0)),
            pl.BlockSpec((tb, D), lambda i, e: (i, 0)),
            pl.BlockSpec((1, D, F), lambda i, e: (e, 0, 0)),
            pl.BlockSpec((1, D, F), lambda i, e: (e, 0, 0)),
            pl.BlockSpec((1, F, D), lambda i, e: (e, 0, 0)),
        ],
        out_specs=pl.BlockSpec((tb, D), lambda i, e: (i, 0)),
        out_shape=jax.ShapeDtypeStruct((t, D), jnp.float32),
        compiler_params=pltpu.CompilerParams(
            dimension_semantics=("arbitrary", "arbitrary")),
    )(comb, xb, gws, uws, dws)

    return y.reshape(orig_shape)


# speedup vs baseline: 1.3994x; 1.3994x over previous
import functools

import jax
import jax.numpy as jnp
from jax import lax
from jax.experimental import pallas as pl
from jax.experimental.pallas import tpu as pltpu
from jax.experimental.pallas import tpu_sc as plsc

E = 8
TOP_K = 2
D = 2048
F = 1408
BG = 256
SB = 512
NC = 2
NS = 16
NW = NC * NS


def _gate_kernel(x_ref, gw_ref, e_ref, w_ref):
    x = x_ref[...]
    logits = jax.lax.dot_general(
        x, gw_ref[...], (((1,), (1,)), ((), ())),
        preferred_element_type=jnp.float32)
    tb = logits.shape[0]
    lane = jax.lax.broadcasted_iota(jnp.int32, (tb, E), 1)
    m1 = jnp.max(logits, axis=1, keepdims=True)
    e1 = jnp.min(jnp.where(logits == m1, lane, E), axis=1, keepdims=True)
    masked = jnp.where(lane == e1, -jnp.inf, logits)
    m2 = jnp.max(masked, axis=1, keepdims=True)
    e2 = jnp.min(jnp.where(masked == m2, lane, E), axis=1, keepdims=True)
    w1 = 1.0 / (1.0 + jnp.exp(m2 - m1))
    e_ref[...] = jnp.concatenate([e1, e2], axis=1)
    w_ref[...] = jnp.concatenate([w1, 1.0 - w1], axis=1)


def _sort_kernel(e_ref, dest_ref, gid_ref, cnt_ref, car_ref, off_ref):
    p = pl.program_id(0)
    b = pl.program_id(1)

    @pl.when((p == 0) & (b == 0))
    def _():
        cnt_ref[...] = jnp.zeros_like(cnt_ref)
        gid_ref[...] = jnp.zeros_like(gid_ref)

    ev = e_ref[0, 0, :].astype(jnp.float32)
    sub = jax.lax.broadcasted_iota(jnp.int32, (E, SB), 0).astype(jnp.float32)
    oh = (jnp.broadcast_to(ev[None, :], (E, SB)) == sub).astype(jnp.float32)
    counts = jnp.sum(oh, axis=1, keepdims=True)

    @pl.when(p == 0)
    def _():
        cnt_ref[:, :1] += counts

    @pl.when((p == 1) & (b == 0))
    def _():
        tot = cnt_ref[:, :1]
        ac = jnp.floor((tot + (BG - 1)) / BG) * BG
        tril = (jax.lax.broadcasted_iota(jnp.int32, (E, E), 1)
                <= jax.lax.broadcasted_iota(jnp.int32, (E, E), 0)
                ).astype(jnp.float32)
        end = jnp.dot(tril, ac, preferred_element_type=jnp.float32)
        off_ref[:, :1] = end - ac
        car_ref[...] = jnp.zeros_like(car_ref)
        g = jax.lax.broadcasted_iota(jnp.int32, (E, 128), 1).astype(
            jnp.float32) * BG
        gid = jnp.sum((jnp.broadcast_to(end, (E, 128)) <= g).astype(
            jnp.float32), axis=0, keepdims=True)
        gid_ref[...] = jnp.minimum(gid, E - 1).astype(jnp.int32)

    @pl.when(p == 1)
    def _():
        s_i = jax.lax.broadcasted_iota(jnp.int32, (SB, SB), 0)
        s_j = jax.lax.broadcasted_iota(jnp.int32, (SB, SB), 1)
        ustrict = (s_i < s_j).astype(jnp.float32)
        pref = jnp.dot(oh, ustrict, preferred_element_type=jnp.float32)
        base = off_ref[:, :1] + car_ref[:, :1]
        dest = jnp.sum(oh * (pref + jnp.broadcast_to(base, (E, SB))),
                       axis=0, keepdims=True)
        dest_ref[...] = dest.astype(jnp.int32)[None]
        car_ref[:, :1] += counts


def _group_mm_kernel(gid_ref, xs_ref, ws_ref, gw_ref, uw_ref, dw_ref, ys_ref):
    x = xs_ref[...].astype(jnp.bfloat16)
    g = jnp.dot(x, gw_ref[0], preferred_element_type=jnp.float32)
    u = jnp.dot(x, uw_ref[0], preferred_element_type=jnp.float32)
    h = (g * jax.lax.logistic(g) * u) * ws_ref[...]
    ys_ref[...] = jnp.dot(h.astype(jnp.bfloat16), dw_ref[0],
                          preferred_element_type=jnp.float32)


def _final_kernel(x_ref, yg_ref, gw_ref, uw_ref, dw_ref, o_ref):
    x = x_ref[...].astype(jnp.bfloat16)
    g = jnp.dot(x, gw_ref[...], preferred_element_type=jnp.float32)
    u = jnp.dot(x, uw_ref[...], preferred_element_type=jnp.float32)
    h = (g * jax.lax.logistic(g) * u).astype(jnp.bfloat16)
    z = jnp.dot(h, dw_ref[...], preferred_element_type=jnp.float32)
    o_ref[...] = z + yg_ref[:, :D] + yg_ref[:, D:]


def _make_scatter_kernel(s, s_pad):
    rows = s // 128 // NW
    mesh = plsc.VectorSubcoreMesh(core_axis_name="c", subcore_axis_name="s")

    @functools.partial(
        pl.kernel, mesh=mesh,
        out_type=[
            jax.ShapeDtypeStruct((s_pad,), jnp.int32),
            jax.ShapeDtypeStruct((s_pad,), jnp.float32),
        ],
        scratch_types=[
            pltpu.VMEM((rows, 128), jnp.int32),
            pltpu.VMEM((rows, 128), jnp.int32),
            pltpu.VMEM((rows, 128), jnp.float32),
            pltpu.SemaphoreType.DMA,
        ],
    )
    def scatter_kernel(dest_hbm, tok_hbm, w_hbm, srctok_hbm, ws_hbm,
                       idx_v, tok_v, w_v, sem):
        wid = lax.axis_index("s") * NC + lax.axis_index("c")
        pltpu.sync_copy(dest_hbm.at[wid], idx_v)
        pltpu.sync_copy(tok_hbm.at[wid], tok_v)
        pltpu.sync_copy(w_hbm.at[wid], w_v)
        for j in range(rows):
            pltpu.async_copy(tok_v.at[j], srctok_hbm.at[idx_v.at[j]],
                             sem).wait()
            pltpu.async_copy(w_v.at[j], ws_hbm.at[idx_v.at[j]], sem).wait()

    return scatter_kernel


def _make_row_gather_kernel(n_idx, table_rows, clamp_hi):
    rows = n_idx // 16 // NW
    chunk = 16
    mesh = plsc.VectorSubcoreMesh(core_axis_name="c", subcore_axis_name="s")

    @functools.partial(
        pl.kernel, mesh=mesh,
        out_type=jax.ShapeDtypeStruct((n_idx, D), jnp.float32),
        scratch_types=[
            pltpu.VMEM((rows, 16), jnp.int32),
            pltpu.VMEM((rows, 16), jnp.int32),
            pltpu.VMEM((chunk, D), jnp.float32),
            pltpu.SemaphoreType.DMA,
        ],
    )
    def gather_kernel(idx_hbm, table_hbm, out_hbm, idx_v, idxc_v, buf, sem):
        wid = lax.axis_index("s") * NC + lax.axis_index("c")
        r0 = wid * rows
        pltpu.sync_copy(idx_hbm.at[wid], idx_v)
        for j in range(rows):
            v = idx_v[j, :]
            idxc_v[j, :] = jnp.minimum(jnp.maximum(v, 0), clamp_hi)
        for j in range(rows):
            pltpu.async_copy(table_hbm.at[idxc_v.at[j]], buf, sem).wait()
            pltpu.sync_copy(buf, out_hbm.at[pl.ds((r0 + j) * chunk, chunk)])

    return gather_kernel


@jax.jit
def kernel(hidden_states, gate_weight, expert_gate_w, expert_up_w,
           expert_down_w, shared_gate_w, shared_up_w, shared_down_w):
    orig_shape = hidden_states.shape
    x = hidden_states.reshape(-1, D)
    t = x.shape[0]
    s = t * TOP_K
    s_pad = s + E * BG
    nblk = s // SB
    tb = min(t, 512)

    topk_e, topk_w = pl.pallas_call(
        _gate_kernel,
        grid=(t // tb,),
        in_specs=[
            pl.BlockSpec((tb, D), lambda i: (i, 0)),
            pl.BlockSpec((E, D), lambda i: (0, 0)),
        ],
        out_specs=[
            pl.BlockSpec((tb, TOP_K), lambda i: (i, 0)),
            pl.BlockSpec((tb, TOP_K), lambda i: (i, 0)),
        ],
        out_shape=[
            jax.ShapeDtypeStruct((t, TOP_K), jnp.int32),
            jax.ShapeDtypeStruct((t, TOP_K), jnp.float32),
        ],
    )(x, gate_weight)

    eflat3d = topk_e.reshape(nblk, 1, SB)
    dest3d, gid128 = pl.pallas_call(
        _sort_kernel,
        grid=(2, nblk),
        in_specs=[pl.BlockSpec((1, 1, SB), lambda p, b: (b, 0, 0))],
        out_specs=[
            pl.BlockSpec((1, 1, SB), lambda p, b: (b, 0, 0)),
            pl.BlockSpec((1, 128), lambda p, b: (0, 0)),
        ],
        out_shape=[
            jax.ShapeDtypeStruct((nblk, 1, SB), jnp.int32),
            jax.ShapeDtypeStruct((1, 128), jnp.int32),
        ],
        scratch_shapes=[
            pltpu.VMEM((E, 128), jnp.float32),
            pltpu.VMEM((E, 128), jnp.float32),
            pltpu.VMEM((E, 128), jnp.float32),
        ],
    )(eflat3d)
    gid = gid128.reshape(128)[:s_pad // BG]

    tokidx = (jnp.arange(s, dtype=jnp.int32) // TOP_K).reshape(
        NW, s // 128 // NW, 128)
    wflat = topk_w.reshape(NW, s // 128 // NW, 128)
    dest_t = dest3d.reshape(NW, s // 128 // NW, 128)

    srctok, ws = _make_scatter_kernel(s, s_pad)(dest_t, tokidx, wflat)
    xs = _make_row_gather_kernel(s_pad, t, t - 1)(
        srctok.reshape(NW, s_pad // 16 // NW, 16), x)

    bf = jnp.bfloat16
    ys = pl.pallas_call(
        _group_mm_kernel,
        grid_spec=pltpu.PrefetchScalarGridSpec(
            num_scalar_prefetch=1,
            grid=(s_pad // BG,),
            in_specs=[
                pl.BlockSpec((BG, D), lambda g, gid_r: (g, 0)),
                pl.BlockSpec((BG, 1), lambda g, gid_r: (g, 0)),
                pl.BlockSpec((1, D, F), lambda g, gid_r: (gid_r[g], 0, 0)),
                pl.BlockSpec((1, D, F), lambda g, gid_r: (gid_r[g], 0, 0)),
                pl.BlockSpec((1, F, D), lambda g, gid_r: (gid_r[g], 0, 0)),
            ],
            out_specs=pl.BlockSpec((BG, D), lambda g, gid_r: (g, 0)),
        ),
        out_shape=jax.ShapeDtypeStruct((s_pad, D), jnp.float32),
        compiler_params=pltpu.CompilerParams(
            dimension_semantics=("arbitrary",)),
    )(gid, xs, ws.reshape(s_pad, 1), expert_gate_w.astype(bf),
      expert_up_w.astype(bf), expert_down_w.astype(bf))

    ygf = _make_row_gather_kernel(s, s_pad, s_pad - 1)(
        dest3d.reshape(NW, s // 16 // NW, 16), ys)
    yg = ygf.reshape(t, TOP_K * D)

    tb2 = min(t, 256)
    y = pl.pallas_call(
        _final_kernel,
        grid=(t // tb2,),
        in_specs=[
            pl.BlockSpec((tb2, D), lambda i: (i, 0)),
            pl.BlockSpec((tb2, TOP_K * D), lambda i: (i, 0)),
            pl.BlockSpec((D, TOP_K * F), lambda i: (0, 0)),
            pl.BlockSpec((D, TOP_K * F), lambda i: (0, 0)),
            pl.BlockSpec((TOP_K * F, D), lambda i: (0, 0)),
        ],
        out_specs=pl.BlockSpec((tb2, D), lambda i: (i, 0)),
        out_shape=jax.ShapeDtypeStruct((t, D), jnp.float32),
    )(x, yg, shared_gate_w.astype(bf), shared_up_w.astype(bf),
      shared_down_w.astype(bf))

    return y.reshape(orig_shape)
